# baseline (speedup 1.0000x reference)
import jax
import jax.numpy as jnp
from jax import lax
from jax.experimental import pallas as pl
from jax.experimental.pallas import tpu as pltpu
from jax.experimental.pallas import tpu_sc as plsc

N = 10000
E = 320000
D = 128
NUM_CORES = 2
NUM_SUBCORES = 16
NUM_WORKERS = NUM_CORES * NUM_SUBCORES
EPW = E // NUM_WORKERS
CHUNK = 80
NCHUNKS = EPW // CHUNK
NP = 10240
ROWS_PER_TILE = NP // NUM_SUBCORES
BLK_N = 400
NBUF = 2


def _mm_tanh_body(x_ref, w_ref, o_ref):
    o_ref[...] = jnp.tanh(
        jnp.dot(x_ref[...], w_ref[...], preferred_element_type=jnp.float32)
    )


def _mm_tanh(x, w):
    return pl.pallas_call(
        _mm_tanh_body,
        grid=(N // BLK_N,),
        in_specs=[
            pl.BlockSpec((BLK_N, D), lambda i: (i, 0)),
            pl.BlockSpec((D, D), lambda i: (0, 0)),
        ],
        out_specs=pl.BlockSpec((BLK_N, D), lambda i: (i, 0)),
        out_shape=jax.ShapeDtypeStruct((N, D), jnp.float32),
    )(x, w)


def _combine_body(p_ref, o_ref):
    o_ref[...] = p_ref[0] + p_ref[1]


def _combine(p):
    return pl.pallas_call(
        _combine_body,
        grid=(N // BLK_N,),
        in_specs=[pl.BlockSpec((NUM_CORES, BLK_N, D), lambda i: (0, i, 0))],
        out_specs=pl.BlockSpec((BLK_N, D), lambda i: (i, 0)),
        out_shape=jax.ShapeDtypeStruct((N, D), jnp.float32),
    )(p)


def _combine_bias_body(p_ref, b_ref, o_ref):
    o_ref[...] = p_ref[0] + p_ref[1] + b_ref[...]


def _combine_bias(p, b2d):
    return pl.pallas_call(
        _combine_bias_body,
        grid=(N // BLK_N,),
        in_specs=[
            pl.BlockSpec((NUM_CORES, BLK_N, D), lambda i: (0, i, 0)),
            pl.BlockSpec((1, D), lambda i: (0, 0)),
        ],
        out_specs=pl.BlockSpec((BLK_N, D), lambda i: (i, 0)),
        out_shape=jax.ShapeDtypeStruct((N, D), jnp.float32),
    )(p, b2d)


def _unpack_src(packed_v, k, dst_ref):
    sh = jnp.full((16,), 16, jnp.int32)
    for g in range(CHUNK // 16):
        p16 = packed_v[pl.ds(k * CHUNK + g * 16, 16)]
        dst_ref[pl.ds(g * 16, 16)] = lax.shift_right_logical(p16, sh)


def _unpack_dst(packed_v, k, dst_ref):
    mask = jnp.full((16,), 0xFFFF, jnp.int32)
    for g in range(CHUNK // 16):
        p16 = packed_v[pl.ds(k * CHUNK + g * 16, 16)]
        dst_ref[pl.ds(g * 16, 16)] = lax.bitwise_and(p16, mask)


def _spmm_body(x_hbm, packed_hbm, vals_hbm, out_hbm,
               packed_v,
               r0b, r1b, v0b, v1b, si0, si1, di0, sbuf,
               acc_sh, s0, s1, ssem):
    rows = [r0b, r1b]
    vbuf = [v0b, v1b]
    sidx = [si0, si1]
    sems = [s0, s1]
    cid = lax.axis_index("c")
    sid = lax.axis_index("s")
    wid = sid * NUM_CORES + cid
    row0 = sid * ROWS_PER_TILE

    pltpu.sync_copy(packed_hbm.at[wid], packed_v)

    def zrow(i, c):
        for r in range(D // 16):
            r0b[i, pl.ds(r * 16, 16)] = jnp.zeros((16,), jnp.float32)
        return c
    lax.fori_loop(0, CHUNK, zrow, 0)
    for j in range(ROWS_PER_TILE // CHUNK):
        pltpu.sync_copy(r0b, acc_sh.at[pl.ds(row0 + j * CHUNK, CHUNK)])

    for b in range(NBUF):
        _unpack_src(packed_v, b, sidx[b])
        pltpu.async_copy(x_hbm.at[sidx[b]], rows[b], sems[b])
        pltpu.async_copy(vals_hbm.at[wid, b], vbuf[b], sems[b])

    plsc.subcore_barrier()

    def chunk_step(k, b, first=False, refill=True):
        pltpu.make_async_copy(x_hbm.at[sidx[b]], rows[b], sems[b]).wait()
        pltpu.make_async_copy(vals_hbm.at[wid, k], vbuf[b],
                              sems[b]).wait()
        if not first:
            pltpu.make_async_copy(sbuf, acc_sh.at[di0], ssem).wait()
        _unpack_dst(packed_v, k, di0)

        def sgrp(g2, c2, _rows=rows[b], _vb=vbuf[b]):
            vv = _vb[0, pl.ds(g2 * 16, 16)]
            for j in range(16):
                e = g2 * 16 + j
                v = vv[j]
                for r in range(D // 16):
                    sl = pl.ds(r * 16, 16)
                    sbuf[e, sl] = _rows[e, sl] * v
            return c2
        lax.fori_loop(0, CHUNK // 16, sgrp, 0)

        pltpu.async_copy(sbuf, acc_sh.at[di0], ssem, add=True)

        if refill:
            kn = k + NBUF

            @pl.when(kn < NCHUNKS)
            def _():
                _unpack_src(packed_v, kn, sidx[b])
                pltpu.async_copy(x_hbm.at[sidx[b]], rows[b], sems[b])
                pltpu.async_copy(vals_hbm.at[wid, kn], vbuf[b], sems[b])

    chunk_step(0, 0, first=True)

    def giter(g, c):
        chunk_step(2 * g + 1, 1)
        chunk_step(2 * g + 2, 0)
        return c
    lax.fori_loop(0, (NCHUNKS - 1) // NBUF, giter, 0)

    pltpu.make_async_copy(sbuf, acc_sh.at[di0], ssem).wait()

    plsc.subcore_barrier()
    pltpu.sync_copy(acc_sh.at[pl.ds(row0, ROWS_PER_TILE)],
                    out_hbm.at[cid, pl.ds(row0, ROWS_PER_TILE)])


_spmm = pl.kernel(
    _spmm_body,
    out_type=jax.ShapeDtypeStruct((NUM_CORES, NP, D), jnp.float32),
    mesh=plsc.VectorSubcoreMesh(core_axis_name="c", subcore_axis_name="s"),
    scratch_types=[
        pltpu.VMEM((EPW,), jnp.int32),
        pltpu.VMEM((CHUNK, D), jnp.float32),
        pltpu.VMEM((CHUNK, D), jnp.float32),
        pltpu.VMEM((1, CHUNK), jnp.float32),
        pltpu.VMEM((1, CHUNK), jnp.float32),
        pltpu.VMEM((CHUNK,), jnp.int32),
        pltpu.VMEM((CHUNK,), jnp.int32),
        pltpu.VMEM((CHUNK,), jnp.int32),
        pltpu.VMEM((CHUNK, D), jnp.float32),
        pltpu.VMEM_SHARED((NP, D), jnp.float32),
        pltpu.SemaphoreType.DMA,
        pltpu.SemaphoreType.DMA,
        pltpu.SemaphoreType.DMA,
    ],
)


@jax.jit
def kernel(features, adj_indices, adj_values, W, b):
    dst = adj_indices[0]
    src = adj_indices[1]
    packed = (src * 65536 + dst).reshape(NUM_WORKERS, EPW)
    vals = adj_values.reshape(NUM_WORKERS, NCHUNKS, 1, CHUNK)
    h = _mm_tanh(features, W)
    p1 = _spmm(h, packed, vals)
    h1 = _combine(p1)
    p2 = _spmm(h1, packed, vals)
    return _combine_bias(p2, b.reshape(1, D))


# speedup vs baseline: 1.1923x; 1.1923x over previous
import jax
import jax.numpy as jnp
from jax import lax
from jax.experimental import pallas as pl
from jax.experimental.pallas import tpu as pltpu
from jax.experimental.pallas import tpu_sc as plsc

N = 10000
E = 320000
D = 128
NUM_CORES = 2
NUM_SUBCORES = 16
NUM_WORKERS = NUM_CORES * NUM_SUBCORES
EPW = E // NUM_WORKERS
CHUNK = 80
NCHUNKS = EPW // CHUNK
NP = 10240
ROWS_PER_TILE = NP // NUM_SUBCORES
BLK_N = 400
NBUF = 2


def _mm_tanh_body(x_ref, w_ref, o_ref):
    o_ref[...] = jnp.tanh(
        jnp.dot(x_ref[...], w_ref[...], preferred_element_type=jnp.float32)
    )


def _mm_tanh(x, w):
    return pl.pallas_call(
        _mm_tanh_body,
        grid=(N // BLK_N,),
        in_specs=[
            pl.BlockSpec((BLK_N, D), lambda i: (i, 0)),
            pl.BlockSpec((D, D), lambda i: (0, 0)),
        ],
        out_specs=pl.BlockSpec((BLK_N, D), lambda i: (i, 0)),
        out_shape=jax.ShapeDtypeStruct((N, D), jnp.float32),
    )(x, w)


def _combine_body(p_ref, o_ref):
    o_ref[...] = p_ref[0] + p_ref[1]


def _combine(p):
    return pl.pallas_call(
        _combine_body,
        grid=(N // BLK_N,),
        in_specs=[pl.BlockSpec((NUM_CORES, BLK_N, D), lambda i: (0, i, 0))],
        out_specs=pl.BlockSpec((BLK_N, D), lambda i: (i, 0)),
        out_shape=jax.ShapeDtypeStruct((N, D), jnp.float32),
    )(p)


def _combine_bias_body(p_ref, b_ref, o_ref):
    o_ref[...] = p_ref[0] + p_ref[1] + b_ref[...]


def _combine_bias(p, b2d):
    return pl.pallas_call(
        _combine_bias_body,
        grid=(N // BLK_N,),
        in_specs=[
            pl.BlockSpec((NUM_CORES, BLK_N, D), lambda i: (0, i, 0)),
            pl.BlockSpec((1, D), lambda i: (0, 0)),
        ],
        out_specs=pl.BlockSpec((BLK_N, D), lambda i: (i, 0)),
        out_shape=jax.ShapeDtypeStruct((N, D), jnp.float32),
    )(p, b2d)


def _unpack_src(packed_v, k, dst_ref):
    sh = jnp.full((16,), 16, jnp.int32)
    for g in range(CHUNK // 16):
        p16 = packed_v[pl.ds(k * CHUNK + g * 16, 16)]
        dst_ref[pl.ds(g * 16, 16)] = lax.shift_right_logical(p16, sh)


def _unpack_dst(packed_v, k, dst_ref):
    mask = jnp.full((16,), 0xFFFF, jnp.int32)
    for g in range(CHUNK // 16):
        p16 = packed_v[pl.ds(k * CHUNK + g * 16, 16)]
        dst_ref[pl.ds(g * 16, 16)] = lax.bitwise_and(p16, mask)


def _spmm_body(x_hbm, packed_hbm, vals_hbm, out_hbm,
               packed_v,
               r0b, r1b, v0b, v1b, si0, si1, di0, sbuf,
               acc_sh, s0, s1, ssem):
    rows = [r0b, r1b]
    vbuf = [v0b, v1b]
    sidx = [si0, si1]
    sems = [s0, s1]
    cid = lax.axis_index("c")
    sid = lax.axis_index("s")
    wid = sid * NUM_CORES + cid
    row0 = sid * ROWS_PER_TILE

    pltpu.sync_copy(packed_hbm.at[wid], packed_v)

    def zrow(i, c):
        for r in range(D // 16):
            r0b[i, pl.ds(r * 16, 16)] = jnp.zeros((16,), jnp.float32)
        return c
    lax.fori_loop(0, CHUNK, zrow, 0)
    for j in range(ROWS_PER_TILE // CHUNK):
        pltpu.sync_copy(r0b, acc_sh.at[pl.ds(row0 + j * CHUNK, CHUNK)])

    for b in range(NBUF):
        _unpack_src(packed_v, b, sidx[b])
        pltpu.async_copy(x_hbm.at[sidx[b]], rows[b], sems[b])
        pltpu.async_copy(vals_hbm.at[wid, b], vbuf[b], sems[b])

    plsc.subcore_barrier()

    def chunk_step(k, b, first=False, refill=True):
        pltpu.make_async_copy(x_hbm.at[sidx[b]], rows[b], sems[b]).wait()
        pltpu.make_async_copy(vals_hbm.at[wid, k], vbuf[b],
                              sems[b]).wait()
        if not first:
            pltpu.make_async_copy(sbuf, acc_sh.at[di0], ssem).wait()
        _unpack_dst(packed_v, k, di0)

        def sgrp(g2, c2, _rows=rows[b], _vb=vbuf[b]):
            vv = _vb[0, pl.ds(g2 * 16, 16)]
            for j in range(16):
                e = g2 * 16 + j
                v = vv[j]
                for r in range(D // 16):
                    sl = pl.ds(r * 16, 16)
                    sbuf[e, sl] = _rows[e, sl] * v
            return c2
        lax.fori_loop(0, CHUNK // 16, sgrp, 0, unroll=True)

        if refill:
            kn = k + NBUF

            @pl.when(kn < NCHUNKS)
            def _():
                _unpack_src(packed_v, kn, sidx[b])
                pltpu.async_copy(x_hbm.at[sidx[b]], rows[b], sems[b])
                pltpu.async_copy(vals_hbm.at[wid, kn], vbuf[b], sems[b])

        pltpu.async_copy(sbuf, acc_sh.at[di0], ssem, add=True)

    chunk_step(0, 0, first=True)

    def giter(g, c):
        chunk_step(2 * g + 1, 1)
        chunk_step(2 * g + 2, 0)
        return c
    lax.fori_loop(0, (NCHUNKS - 1) // NBUF, giter, 0)

    pltpu.make_async_copy(sbuf, acc_sh.at[di0], ssem).wait()

    plsc.subcore_barrier()
    pltpu.sync_copy(acc_sh.at[pl.ds(row0, ROWS_PER_TILE)],
                    out_hbm.at[cid, pl.ds(row0, ROWS_PER_TILE)])


_spmm = pl.kernel(
    _spmm_body,
    out_type=jax.ShapeDtypeStruct((NUM_CORES, NP, D), jnp.float32),
    mesh=plsc.VectorSubcoreMesh(core_axis_name="c", subcore_axis_name="s"),
    scratch_types=[
        pltpu.VMEM((EPW,), jnp.int32),
        pltpu.VMEM((CHUNK, D), jnp.float32),
        pltpu.VMEM((CHUNK, D), jnp.float32),
        pltpu.VMEM((1, CHUNK), jnp.float32),
        pltpu.VMEM((1, CHUNK), jnp.float32),
        pltpu.VMEM((CHUNK,), jnp.int32),
        pltpu.VMEM((CHUNK,), jnp.int32),
        pltpu.VMEM((CHUNK,), jnp.int32),
        pltpu.VMEM((CHUNK, D), jnp.float32),
        pltpu.VMEM_SHARED((NP, D), jnp.float32),
        pltpu.SemaphoreType.DMA,
        pltpu.SemaphoreType.DMA,
        pltpu.SemaphoreType.DMA,
    ],
)


@jax.jit
def kernel(features, adj_indices, adj_values, W, b):
    dst = adj_indices[0]
    src = adj_indices[1]
    packed = (src * 65536 + dst).reshape(NUM_WORKERS, EPW)
    vals = adj_values.reshape(NUM_WORKERS, NCHUNKS, 1, CHUNK)
    h = _mm_tanh(features, W)
    p1 = _spmm(h, packed, vals)
    h1 = _combine(p1)
    p2 = _spmm(h1, packed, vals)
    return _combine_bias(p2, b.reshape(1, D))


# speedup vs baseline: 1.3126x; 1.1008x over previous
import jax
import jax.numpy as jnp
from jax import lax
from jax.experimental import pallas as pl
from jax.experimental.pallas import tpu as pltpu
from jax.experimental.pallas import tpu_sc as plsc

N = 10000
E = 320000
D = 128
NUM_CORES = 2
NUM_SUBCORES = 16
NUM_WORKERS = NUM_CORES * NUM_SUBCORES
EPW = E // NUM_WORKERS
CHUNK = 80
NCHUNKS = EPW // CHUNK
NP = 10240
ROWS_PER_TILE = NP // NUM_SUBCORES
BLK_N = 2000
NBUF = 2


def _mm_tanh_body(x_ref, w_ref, o_ref):
    o_ref[...] = jnp.tanh(
        jnp.dot(x_ref[...], w_ref[...], preferred_element_type=jnp.float32)
    )


def _mm_tanh(x, w):
    return pl.pallas_call(
        _mm_tanh_body,
        grid=(N // BLK_N,),
        in_specs=[
            pl.BlockSpec((BLK_N, D), lambda i: (i, 0)),
            pl.BlockSpec((D, D), lambda i: (0, 0)),
        ],
        out_specs=pl.BlockSpec((BLK_N, D), lambda i: (i, 0)),
        out_shape=jax.ShapeDtypeStruct((N, D), jnp.float32),
    )(x, w)


def _combine_body(p_ref, o_ref):
    o_ref[...] = p_ref[0] + p_ref[1]


def _combine(p):
    return pl.pallas_call(
        _combine_body,
        grid=(N // BLK_N,),
        in_specs=[pl.BlockSpec((NUM_CORES, BLK_N, D), lambda i: (0, i, 0))],
        out_specs=pl.BlockSpec((BLK_N, D), lambda i: (i, 0)),
        out_shape=jax.ShapeDtypeStruct((N, D), jnp.float32),
    )(p)


def _combine_bias_body(p_ref, b_ref, o_ref):
    o_ref[...] = p_ref[0] + p_ref[1] + b_ref[...]


def _combine_bias(p, b2d):
    return pl.pallas_call(
        _combine_bias_body,
        grid=(N // BLK_N,),
        in_specs=[
            pl.BlockSpec((NUM_CORES, BLK_N, D), lambda i: (0, i, 0)),
            pl.BlockSpec((1, D), lambda i: (0, 0)),
        ],
        out_specs=pl.BlockSpec((BLK_N, D), lambda i: (i, 0)),
        out_shape=jax.ShapeDtypeStruct((N, D), jnp.float32),
    )(p, b2d)


def _unpack_src(packed_v, k, dst_ref):
    sh = jnp.full((16,), 16, jnp.int32)
    for g in range(CHUNK // 16):
        p16 = packed_v[pl.ds(k * CHUNK + g * 16, 16)]
        dst_ref[pl.ds(g * 16, 16)] = lax.shift_right_logical(p16, sh)


def _unpack_dst(packed_v, k, dst_ref):
    mask = jnp.full((16,), 0xFFFF, jnp.int32)
    for g in range(CHUNK // 16):
        p16 = packed_v[pl.ds(k * CHUNK + g * 16, 16)]
        dst_ref[pl.ds(g * 16, 16)] = lax.bitwise_and(p16, mask)


def _spmm_body(x_hbm, packed_hbm, vals_hbm, out_hbm,
               packed_v,
               r0b, r1b, v0b, v1b, si0, si1, di0, sbuf,
               acc_sh, s0, s1, ssem):
    rows = [r0b, r1b]
    vbuf = [v0b, v1b]
    sidx = [si0, si1]
    sems = [s0, s1]
    cid = lax.axis_index("c")
    sid = lax.axis_index("s")
    wid = sid * NUM_CORES + cid
    row0 = sid * ROWS_PER_TILE

    pltpu.sync_copy(packed_hbm.at[wid], packed_v)

    for b in range(NBUF):
        _unpack_src(packed_v, b, sidx[b])
        pltpu.async_copy(x_hbm.at[sidx[b]], rows[b], sems[b])
        pltpu.async_copy(vals_hbm.at[wid, b], vbuf[b], sems[b])

    def zrow(i, c):
        for r in range(D // 16):
            sbuf[i, pl.ds(r * 16, 16)] = jnp.zeros((16,), jnp.float32)
        return c
    lax.fori_loop(0, CHUNK, zrow, 0)
    for j in range(ROWS_PER_TILE // CHUNK):
        pltpu.sync_copy(sbuf, acc_sh.at[pl.ds(row0 + j * CHUNK, CHUNK)])

    plsc.subcore_barrier()

    def chunk_step(k, b, first=False, refill=True):
        pltpu.make_async_copy(x_hbm.at[sidx[b]], rows[b], sems[b]).wait()
        pltpu.make_async_copy(vals_hbm.at[wid, k], vbuf[b],
                              sems[b]).wait()
        if not first:
            pltpu.make_async_copy(sbuf, acc_sh.at[di0], ssem).wait()
        _unpack_dst(packed_v, k, di0)

        def sgrp(g2, c2, _rows=rows[b], _vb=vbuf[b]):
            vv = _vb[0, pl.ds(g2 * 16, 16)]
            for j in range(16):
                e = g2 * 16 + j
                v = vv[j]
                for r in range(D // 16):
                    sl = pl.ds(r * 16, 16)
                    sbuf[e, sl] = _rows[e, sl] * v
            return c2
        lax.fori_loop(0, CHUNK // 16, sgrp, 0, unroll=True)

        if refill:
            kn = k + NBUF

            @pl.when(kn < NCHUNKS)
            def _():
                _unpack_src(packed_v, kn, sidx[b])
                pltpu.async_copy(x_hbm.at[sidx[b]], rows[b], sems[b])
                pltpu.async_copy(vals_hbm.at[wid, kn], vbuf[b], sems[b])

        pltpu.async_copy(sbuf, acc_sh.at[di0], ssem, add=True)

    chunk_step(0, 0, first=True)

    def giter(g, c):
        chunk_step(2 * g + 1, 1)
        chunk_step(2 * g + 2, 0)
        return c
    lax.fori_loop(0, (NCHUNKS - 1) // NBUF, giter, 0)

    pltpu.make_async_copy(sbuf, acc_sh.at[di0], ssem).wait()

    plsc.subcore_barrier()
    pltpu.sync_copy(acc_sh.at[pl.ds(row0, ROWS_PER_TILE)],
                    out_hbm.at[cid, pl.ds(row0, ROWS_PER_TILE)])


_spmm = pl.kernel(
    _spmm_body,
    out_type=jax.ShapeDtypeStruct((NUM_CORES, NP, D), jnp.float32),
    mesh=plsc.VectorSubcoreMesh(core_axis_name="c", subcore_axis_name="s"),
    scratch_types=[
        pltpu.VMEM((EPW,), jnp.int32),
        pltpu.VMEM((CHUNK, D), jnp.float32),
        pltpu.VMEM((CHUNK, D), jnp.float32),
        pltpu.VMEM((1, CHUNK), jnp.float32),
        pltpu.VMEM((1, CHUNK), jnp.float32),
        pltpu.VMEM((CHUNK,), jnp.int32),
        pltpu.VMEM((CHUNK,), jnp.int32),
        pltpu.VMEM((CHUNK,), jnp.int32),
        pltpu.VMEM((CHUNK, D), jnp.float32),
        pltpu.VMEM_SHARED((NP, D), jnp.float32),
        pltpu.SemaphoreType.DMA,
        pltpu.SemaphoreType.DMA,
        pltpu.SemaphoreType.DMA,
    ],
)


@jax.jit
def kernel(features, adj_indices, adj_values, W, b):
    dst = adj_indices[0]
    src = adj_indices[1]
    packed = (src * 65536 + dst).reshape(NUM_WORKERS, EPW)
    vals = adj_values.reshape(NUM_WORKERS, NCHUNKS, 1, CHUNK)
    h = _mm_tanh(features, W)
    p1 = _spmm(h, packed, vals)
    h1 = _combine(p1)
    p2 = _spmm(h1, packed, vals)
    return _combine_bias(p2, b.reshape(1, D))


# speedup vs baseline: 1.3939x; 1.0620x over previous
import jax
import jax.numpy as jnp
from jax import lax
from jax.experimental import pallas as pl
from jax.experimental.pallas import tpu as pltpu
from jax.experimental.pallas import tpu_sc as plsc

N = 10000
E = 320000
D = 128
NUM_CORES = 2
NUM_SUBCORES = 16
NUM_WORKERS = NUM_CORES * NUM_SUBCORES
EPW = E // NUM_WORKERS
CHUNK = 80
NCHUNKS = EPW // CHUNK
NP = 10240
ROWS_PER_TILE = NP // NUM_SUBCORES
BLK_N = 2000
NBUF = 2


def _mm_tanh_body(x_ref, w_ref, o_ref):
    o_ref[...] = jnp.tanh(
        jnp.dot(x_ref[...], w_ref[...], preferred_element_type=jnp.float32)
    )


def _mm_tanh(x, w):
    return pl.pallas_call(
        _mm_tanh_body,
        grid=(N // BLK_N,),
        in_specs=[
            pl.BlockSpec((BLK_N, D), lambda i: (i, 0)),
            pl.BlockSpec((D, D), lambda i: (0, 0)),
        ],
        out_specs=pl.BlockSpec((BLK_N, D), lambda i: (i, 0)),
        out_shape=jax.ShapeDtypeStruct((N, D), jnp.float32),
    )(x, w)


def _combine_body(p_ref, o_ref):
    o_ref[...] = p_ref[0] + p_ref[1]


def _combine(p):
    return pl.pallas_call(
        _combine_body,
        grid=(N // BLK_N,),
        in_specs=[pl.BlockSpec((NUM_CORES, BLK_N, D), lambda i: (0, i, 0))],
        out_specs=pl.BlockSpec((BLK_N, D), lambda i: (i, 0)),
        out_shape=jax.ShapeDtypeStruct((N, D), jnp.float32),
    )(p)


def _combine_bias_body(p_ref, b_ref, o_ref):
    o_ref[...] = p_ref[0] + p_ref[1] + b_ref[...]


def _combine_bias(p, b2d):
    return pl.pallas_call(
        _combine_bias_body,
        grid=(N // BLK_N,),
        in_specs=[
            pl.BlockSpec((NUM_CORES, BLK_N, D), lambda i: (0, i, 0)),
            pl.BlockSpec((1, D), lambda i: (0, 0)),
        ],
        out_specs=pl.BlockSpec((BLK_N, D), lambda i: (i, 0)),
        out_shape=jax.ShapeDtypeStruct((N, D), jnp.float32),
    )(p, b2d)


def _unpack_src(packed_v, k, dst_ref):
    sh = jnp.full((16,), 16, jnp.int32)
    for g in range(CHUNK // 16):
        p16 = packed_v[pl.ds(k * CHUNK + g * 16, 16)]
        dst_ref[pl.ds(g * 16, 16)] = lax.shift_right_logical(p16, sh)


def _unpack_dst_half(packed_v, k, half, dst_ref):
    mask = jnp.full((16,), 0xFFFF, jnp.int32)
    base = k * CHUNK + half * (CHUNK // 2)
    for off in (0, 16, 24):
        p16 = packed_v[pl.ds(base + off, 16)]
        dst_ref[pl.ds(off, 16)] = lax.bitwise_and(p16, mask)


def _spmm_body(x_hbm, packed_hbm, vals_hbm, out_hbm,
               packed_v,
               r0b, r1b, v0b, v1b, si0, si1, dia, dib, sbufa, sbufb,
               acc_sh, s0, s1, ssema, ssemb):
    rows = [r0b, r1b]
    vbuf = [v0b, v1b]
    sidx = [si0, si1]
    sems = [s0, s1]
    HC = CHUNK // 2
    cid = lax.axis_index("c")
    sid = lax.axis_index("s")
    wid = sid * NUM_CORES + cid
    row0 = sid * ROWS_PER_TILE

    pltpu.sync_copy(packed_hbm.at[wid], packed_v)

    _unpack_src(packed_v, 0, sidx[0])
    pltpu.async_copy(x_hbm.at[sidx[0]], rows[0], sems[0])
    pltpu.async_copy(vals_hbm.at[wid, 0], vbuf[0], sems[0])

    def zrow(i, c):
        for r in range(D // 16):
            r1b[i, pl.ds(r * 16, 16)] = jnp.zeros((16,), jnp.float32)
        return c
    lax.fori_loop(0, CHUNK, zrow, 0)
    for j in range(ROWS_PER_TILE // CHUNK):
        pltpu.sync_copy(r1b, acc_sh.at[pl.ds(row0 + j * CHUNK, CHUNK)])

    _unpack_src(packed_v, 1, sidx[1])
    pltpu.async_copy(x_hbm.at[sidx[1]], rows[1], sems[1])
    pltpu.async_copy(vals_hbm.at[wid, 1], vbuf[1], sems[1])

    plsc.subcore_barrier()

    def scale_half(rbuf, vb, sb, half):
        for g2, lane0, n in ((0, 0, 16), (1, 0, 16), (2, 8, 8)):
            off = half * HC + g2 * 16 - (8 if g2 == 2 else 0)
            vv = vb[0, pl.ds(off, 16)]
            for j in range(n):
                e = g2 * 16 + j
                v = vv[lane0 + j]
                for r in range(D // 16):
                    sl = pl.ds(r * 16, 16)
                    sb[e, sl] = rbuf[half * HC + e, sl] * v

    def chunk_step(k, b, first=False, refill=True):
        pltpu.make_async_copy(x_hbm.at[sidx[b]], rows[b], sems[b]).wait()
        pltpu.make_async_copy(vals_hbm.at[wid, k], vbuf[b],
                              sems[b]).wait()
        if not first:
            pltpu.make_async_copy(sbufa, acc_sh.at[dia], ssema).wait()
        _unpack_dst_half(packed_v, k, 0, dia)
        scale_half(rows[b], vbuf[b], sbufa, 0)
        pltpu.async_copy(sbufa, acc_sh.at[dia], ssema, add=True)

        if not first:
            pltpu.make_async_copy(sbufb, acc_sh.at[dib], ssemb).wait()
        _unpack_dst_half(packed_v, k, 1, dib)
        scale_half(rows[b], vbuf[b], sbufb, 1)

        if refill:
            kn = k + NBUF

            @pl.when(kn < NCHUNKS)
            def _():
                _unpack_src(packed_v, kn, sidx[b])
                pltpu.async_copy(x_hbm.at[sidx[b]], rows[b], sems[b])
                pltpu.async_copy(vals_hbm.at[wid, kn], vbuf[b], sems[b])

        pltpu.async_copy(sbufb, acc_sh.at[dib], ssemb, add=True)

    chunk_step(0, 0, first=True)

    def giter(g, c):
        chunk_step(2 * g + 1, 1)
        chunk_step(2 * g + 2, 0)
        return c
    lax.fori_loop(0, (NCHUNKS - 1) // NBUF, giter, 0)

    pltpu.make_async_copy(sbufa, acc_sh.at[dia], ssema).wait()
    pltpu.make_async_copy(sbufb, acc_sh.at[dib], ssemb).wait()

    plsc.subcore_barrier()
    pltpu.sync_copy(acc_sh.at[pl.ds(row0, ROWS_PER_TILE)],
                    out_hbm.at[cid, pl.ds(row0, ROWS_PER_TILE)])


_spmm = pl.kernel(
    _spmm_body,
    out_type=jax.ShapeDtypeStruct((NUM_CORES, NP, D), jnp.float32),
    mesh=plsc.VectorSubcoreMesh(core_axis_name="c", subcore_axis_name="s"),
    scratch_types=[
        pltpu.VMEM((EPW,), jnp.int32),
        pltpu.VMEM((CHUNK, D), jnp.float32),
        pltpu.VMEM((CHUNK, D), jnp.float32),
        pltpu.VMEM((1, CHUNK), jnp.float32),
        pltpu.VMEM((1, CHUNK), jnp.float32),
        pltpu.VMEM((CHUNK,), jnp.int32),
        pltpu.VMEM((CHUNK,), jnp.int32),
        pltpu.VMEM((CHUNK // 2,), jnp.int32),
        pltpu.VMEM((CHUNK // 2,), jnp.int32),
        pltpu.VMEM((CHUNK // 2, D), jnp.float32),
        pltpu.VMEM((CHUNK // 2, D), jnp.float32),
        pltpu.VMEM_SHARED((NP, D), jnp.float32),
        pltpu.SemaphoreType.DMA,
        pltpu.SemaphoreType.DMA,
        pltpu.SemaphoreType.DMA,
        pltpu.SemaphoreType.DMA,
    ],
)


@jax.jit
def kernel(features, adj_indices, adj_values, W, b):
    dst = adj_indices[0]
    src = adj_indices[1]
    packed = (src * 65536 + dst).reshape(NUM_WORKERS, EPW)
    vals = adj_values.reshape(NUM_WORKERS, NCHUNKS, 1, CHUNK)
    h = _mm_tanh(features, W)
    p1 = _spmm(h, packed, vals)
    h1 = _combine(p1)
    p2 = _spmm(h1, packed, vals)
    return _combine_bias(p2, b.reshape(1, D))


# speedup vs baseline: 1.4004x; 1.0046x over previous
import jax
import jax.numpy as jnp
from jax import lax
from jax.experimental import pallas as pl
from jax.experimental.pallas import tpu as pltpu
from jax.experimental.pallas import tpu_sc as plsc

N = 10000
E = 320000
D = 128
NUM_CORES = 2
NUM_SUBCORES = 16
NUM_WORKERS = NUM_CORES * NUM_SUBCORES
EPW = E // NUM_WORKERS
CHUNK = 80
NCHUNKS = EPW // CHUNK
NP = 10240
ROWS_PER_TILE = NP // NUM_SUBCORES
BLK_N = 2000
NBUF = 2


def _mm_tanh_body(x_ref, w_ref, o_ref):
    o_ref[...] = jnp.tanh(
        jnp.dot(x_ref[...], w_ref[...], preferred_element_type=jnp.float32)
    )


def _mm_tanh(x, w):
    return pl.pallas_call(
        _mm_tanh_body,
        grid=(N // BLK_N,),
        in_specs=[
            pl.BlockSpec((BLK_N, D), lambda i: (i, 0)),
            pl.BlockSpec((D, D), lambda i: (0, 0)),
        ],
        out_specs=pl.BlockSpec((BLK_N, D), lambda i: (i, 0)),
        out_shape=jax.ShapeDtypeStruct((N, D), jnp.float32),
    )(x, w)


def _combine_body(p_ref, o_ref):
    o_ref[...] = p_ref[0] + p_ref[1]


def _combine(p):
    return pl.pallas_call(
        _combine_body,
        grid=(N // BLK_N,),
        in_specs=[pl.BlockSpec((NUM_CORES, BLK_N, D), lambda i: (0, i, 0))],
        out_specs=pl.BlockSpec((BLK_N, D), lambda i: (i, 0)),
        out_shape=jax.ShapeDtypeStruct((N, D), jnp.float32),
    )(p)


def _combine_bias_body(p_ref, b_ref, o_ref):
    o_ref[...] = p_ref[0] + p_ref[1] + b_ref[...]


def _combine_bias(p, b2d):
    return pl.pallas_call(
        _combine_bias_body,
        grid=(N // BLK_N,),
        in_specs=[
            pl.BlockSpec((NUM_CORES, BLK_N, D), lambda i: (0, i, 0)),
            pl.BlockSpec((1, D), lambda i: (0, 0)),
        ],
        out_specs=pl.BlockSpec((BLK_N, D), lambda i: (i, 0)),
        out_shape=jax.ShapeDtypeStruct((N, D), jnp.float32),
    )(p, b2d)


def _unpack_src(packed_v, k, dst_ref):
    sh = jnp.full((16,), 16, jnp.int32)
    for g in range(CHUNK // 16):
        p16 = packed_v[pl.ds(k * CHUNK + g * 16, 16)]
        dst_ref[pl.ds(g * 16, 16)] = lax.shift_right_logical(p16, sh)


def _unpack_dst_half(packed_v, k, half, dst_ref):
    mask = jnp.full((16,), 0xFFFF, jnp.int32)
    base = k * CHUNK + half * (CHUNK // 2)
    for off in (0, 16, 24):
        p16 = packed_v[pl.ds(base + off, 16)]
        dst_ref[pl.ds(off, 16)] = lax.bitwise_and(p16, mask)


def _spmm_body(x_hbm, packed_hbm, vals_hbm, out_hbm,
               packed_v,
               r0b, r1b, v0b, v1b, si0, si1, dia, dib, sbufa, sbufb,
               acc_sh, s0, s1, ssema, ssemb):
    rows = [r0b, r1b]
    vbuf = [v0b, v1b]
    sidx = [si0, si1]
    sems = [s0, s1]
    HC = CHUNK // 2
    cid = lax.axis_index("c")
    sid = lax.axis_index("s")
    wid = sid * NUM_CORES + cid
    row0 = sid * ROWS_PER_TILE

    pltpu.sync_copy(packed_hbm.at[wid], packed_v)

    _unpack_src(packed_v, 0, sidx[0])
    pltpu.async_copy(x_hbm.at[sidx[0]], rows[0], sems[0])
    pltpu.async_copy(vals_hbm.at[wid, 0], vbuf[0], sems[0])

    def zrow(i, c):
        for r in range(D // 16):
            r1b[i, pl.ds(r * 16, 16)] = jnp.zeros((16,), jnp.float32)
        return c
    lax.fori_loop(0, CHUNK, zrow, 0)
    for j in range(ROWS_PER_TILE // CHUNK):
        pltpu.sync_copy(r1b, acc_sh.at[pl.ds(row0 + j * CHUNK, CHUNK)])

    _unpack_src(packed_v, 1, sidx[1])
    pltpu.async_copy(x_hbm.at[sidx[1]], rows[1], sems[1])
    pltpu.async_copy(vals_hbm.at[wid, 1], vbuf[1], sems[1])

    plsc.subcore_barrier()

    def scale_half(rbuf, vb, sb, half):
        for g2, lane0, n in ((0, 0, 16), (1, 0, 16), (2, 8, 8)):
            off = half * HC + g2 * 16 - (8 if g2 == 2 else 0)
            vv = vb[0, pl.ds(off, 16)]
            for j in range(n):
                e = g2 * 16 + j
                v = vv[lane0 + j]
                for r in range(D // 16):
                    sl = pl.ds(r * 16, 16)
                    sb[e, sl] = rbuf[half * HC + e, sl] * v

    def chunk_step(k, b, first=False, refill=True):
        pltpu.make_async_copy(x_hbm.at[sidx[b]], rows[b], sems[b]).wait()
        pltpu.make_async_copy(vals_hbm.at[wid, k], vbuf[b],
                              sems[b]).wait()
        if not first:
            pltpu.make_async_copy(sbufa, acc_sh.at[dia], ssema).wait()
        _unpack_dst_half(packed_v, k, 0, dia)
        scale_half(rows[b], vbuf[b], sbufa, 0)
        pltpu.async_copy(sbufa, acc_sh.at[dia], ssema, add=True,
                         priority=1)

        if not first:
            pltpu.make_async_copy(sbufb, acc_sh.at[dib], ssemb).wait()
        _unpack_dst_half(packed_v, k, 1, dib)
        scale_half(rows[b], vbuf[b], sbufb, 1)

        if refill:
            kn = k + NBUF

            @pl.when(kn < NCHUNKS)
            def _():
                _unpack_src(packed_v, kn, sidx[b])
                pltpu.async_copy(x_hbm.at[sidx[b]], rows[b], sems[b])
                pltpu.async_copy(vals_hbm.at[wid, kn], vbuf[b], sems[b])

        pltpu.async_copy(sbufb, acc_sh.at[dib], ssemb, add=True,
                         priority=1)

    chunk_step(0, 0, first=True)

    def giter(g, c):
        chunk_step(2 * g + 1, 1)
        chunk_step(2 * g + 2, 0)
        return c
    lax.fori_loop(0, (NCHUNKS - 1) // NBUF, giter, 0)

    pltpu.make_async_copy(sbufa, acc_sh.at[dia], ssema).wait()
    pltpu.make_async_copy(sbufb, acc_sh.at[dib], ssemb).wait()

    plsc.subcore_barrier()
    pltpu.sync_copy(acc_sh.at[pl.ds(row0, ROWS_PER_TILE)],
                    out_hbm.at[cid, pl.ds(row0, ROWS_PER_TILE)])


_spmm = pl.kernel(
    _spmm_body,
    out_type=jax.ShapeDtypeStruct((NUM_CORES, NP, D), jnp.float32),
    mesh=plsc.VectorSubcoreMesh(core_axis_name="c", subcore_axis_name="s"),
    scratch_types=[
        pltpu.VMEM((EPW,), jnp.int32),
        pltpu.VMEM((CHUNK, D), jnp.float32),
        pltpu.VMEM((CHUNK, D), jnp.float32),
        pltpu.VMEM((1, CHUNK), jnp.float32),
        pltpu.VMEM((1, CHUNK), jnp.float32),
        pltpu.VMEM((CHUNK,), jnp.int32),
        pltpu.VMEM((CHUNK,), jnp.int32),
        pltpu.VMEM((CHUNK // 2,), jnp.int32),
        pltpu.VMEM((CHUNK // 2,), jnp.int32),
        pltpu.VMEM((CHUNK // 2, D), jnp.float32),
        pltpu.VMEM((CHUNK // 2, D), jnp.float32),
        pltpu.VMEM_SHARED((NP, D), jnp.float32),
        pltpu.SemaphoreType.DMA,
        pltpu.SemaphoreType.DMA,
        pltpu.SemaphoreType.DMA,
        pltpu.SemaphoreType.DMA,
    ],
)


@jax.jit
def kernel(features, adj_indices, adj_values, W, b):
    dst = adj_indices[0]
    src = adj_indices[1]
    packed = (src * 65536 + dst).reshape(NUM_WORKERS, EPW)
    vals = adj_values.reshape(NUM_WORKERS, NCHUNKS, 1, CHUNK)
    h = _mm_tanh(features, W)
    p1 = _spmm(h, packed, vals)
    h1 = _combine(p1)
    p2 = _spmm(h1, packed, vals)
    return _combine_bias(p2, b.reshape(1, D))
